# baseline (speedup 1.0000x reference)
import functools

import jax
import jax.numpy as jnp
from jax import lax
from jax.experimental import pallas as pl
from jax.experimental.pallas import tpu as pltpu
from jax.experimental.pallas import tpu_sc as plsc

B = 4
NC = 2500
NF = 10000
CIN = 128
COUT = 128
CMID = 64
E = 320000
NNZ = 30000
G = 32
EPS = 1e-5

NUM_CORES = 2
NUM_SUBCORES = 16
CH = 128
RPS_MAIN = 632
RPS_LAST = NF - (NUM_SUBCORES - 1) * RPS_MAIN


def _cdiv(a, b):
    return (a + b - 1) // b


_GDN = lax.GatherDimensionNumbers(
    offset_dims=(), collapsed_slice_dims=(0,), start_index_map=(0,))


def _splat_lane(vec, e):
    idx = (lax.iota(jnp.int32, 16) * 0 + e).reshape(16, 1)
    return lax.gather(vec, idx, dimension_numbers=_GDN,
                      slice_sizes=(1,),
                      mode=lax.GatherScatterMode.PROMISE_IN_BOUNDS)


def _make_sc_scatter(n_rows_tab, n_rows_out, c, n_edges, n_packs):
    assert n_edges % 8 == 0 and n_edges >= CH
    n_chunks = _ring_chunks(n_edges)
    assert n_chunks % 3 == 2
    e_per_s = n_chunks * CH
    rounds = n_packs // NUM_CORES
    mesh = plsc.VectorSubcoreMesh(core_axis_name="c", subcore_axis_name="s")

    @functools.partial(
        pl.kernel,
        out_type=jax.ShapeDtypeStruct((n_packs * n_rows_out, c), jnp.float32),
        mesh=mesh,
        scratch_types=[
            pltpu.VMEM((3, CH), jnp.int32),
            pltpu.VMEM((3, CH), jnp.int32),
            pltpu.VMEM((3, CH), jnp.float32),
            pltpu.VMEM((3, CH, c), jnp.float32),
            pltpu.VMEM_SHARED((n_rows_out, c), jnp.float32),
            pltpu.SemaphoreType.DMA,
            pltpu.SemaphoreType.DMA,
            pltpu.SemaphoreType.DMA,
            pltpu.SemaphoreType.DMA,
            pltpu.SemaphoreType.DMA,
            pltpu.SemaphoreType.DMA,
            pltpu.SemaphoreType.DMA,
            pltpu.SemaphoreType.DMA,
            pltpu.SemaphoreType.DMA,
        ],
    )
    def sc_kernel(tab_hbm, src_hbm, dst_hbm, val_hbm, out_hbm,
                  sidx_v, didx_v, val_s, msg_v, accum_sh,
                  sg0, sg1, sg2, si0, si1, si2, ss0, ss1, ss2):
        cid = lax.axis_index("c")
        sid = lax.axis_index("s")
        sg = (sg0, sg1, sg2)
        si = (si0, si1, si2)
        ss = (ss0, ss1, ss2)

        zeros16 = jnp.zeros((16,), jnp.float32)

        def zero_msg0(i, _):
            for j in range(c // 16):
                msg_v[0, i, pl.ds(j * 16, 16)] = zeros16
            return 0

        def init_slice(nrows):
            base = sid * RPS_MAIN
            for t in range(_cdiv(nrows, CH)):
                rows = min(CH, nrows - t * CH)
                pltpu.sync_copy(
                    msg_v.at[0].at[pl.ds(0, rows)],
                    accum_sh.at[pl.ds(base + t * CH, rows)])

        def chunk_base(k):
            off = sid * e_per_s + k * CH
            return off, jnp.minimum(off, n_edges - CH)

        def issue_idx(b, k, p):
            del b
            _, bc = chunk_base(k)
            pltpu.async_copy(src_hbm.at[pl.ds(bc, CH)], sidx_v.at[p], si[p])
            pltpu.async_copy(dst_hbm.at[pl.ds(bc, CH)], didx_v.at[p], si[p])
            pltpu.async_copy(val_hbm.at[pl.ds(bc, CH)], val_s.at[p], si[p])

        def wait_idx(b, k, p):
            del b
            _, bc = chunk_base(k)
            pltpu.make_async_copy(src_hbm.at[pl.ds(bc, CH)],
                                  sidx_v.at[p], si[p]).wait()
            pltpu.make_async_copy(dst_hbm.at[pl.ds(bc, CH)],
                                  didx_v.at[p], si[p]).wait()
            pltpu.make_async_copy(val_hbm.at[pl.ds(bc, CH)],
                                  val_s.at[p], si[p]).wait()

        def adjust_src(b, p):
            offv = b * n_rows_tab
            for g in range(CH // 16):
                sidx_v[p, pl.ds(g * 16, 16)] = (
                    sidx_v[p, pl.ds(g * 16, 16)] + offv)

        def issue_gather(p):
            pltpu.async_copy(tab_hbm.at[sidx_v.at[p]], msg_v.at[p], sg[p])

        def wait_gather(p):
            pltpu.make_async_copy(tab_hbm.at[sidx_v.at[p]], msg_v.at[p],
                                  sg[p]).wait()

        def issue_scatter(p):
            pltpu.async_copy(msg_v.at[p], accum_sh.at[didx_v.at[p]], ss[p],
                             add=True)

        def wait_scatter(p):
            pltpu.make_async_copy(msg_v.at[p], accum_sh.at[didx_v.at[p]],
                                  ss[p]).wait()

        def scale(p, k):
            off, bc = chunk_base(k)
            head = off - bc

            def body(g2, _):
                valv = val_s[p, pl.ds(g2 * 16, 16)]
                lane = lax.iota(jnp.int32, 16) + (g2 * 16 - head)
                valv = jnp.where(lane >= 0, valv, 0.0)
                for e2 in range(16):
                    v = _splat_lane(valv, e2)
                    row = g2 * 16 + e2
                    for j in range(c // 16):
                        msg_v[p, row, pl.ds(j * 16, 16)] = (
                            msg_v[p, row, pl.ds(j * 16, 16)] * v)
                return 0
            lax.fori_loop(0, CH // 16, body, 0, unroll=4)

        for r in range(rounds):
            b = cid + NUM_CORES * r

            lax.fori_loop(0, CH, zero_msg0, 0)
            pl.when(sid < NUM_SUBCORES - 1)(
                lambda: init_slice(RPS_MAIN))
            pl.when(sid == NUM_SUBCORES - 1)(
                lambda: init_slice(RPS_LAST))
            plsc.subcore_barrier()

            issue_idx(b, 0, 0)
            issue_idx(b, 1, 1)
            wait_idx(b, 0, 0)
            adjust_src(b, 0)
            issue_gather(0)

            wait_gather(0)
            wait_idx(b, 1, 1)
            adjust_src(b, 1)
            issue_gather(1)
            scale(0, 0)
            issue_scatter(0)
            issue_idx(b, 2, 2)

            wait_gather(1)
            wait_idx(b, 2, 2)
            adjust_src(b, 2)
            issue_gather(2)
            scale(1, 1)
            wait_scatter(0)
            issue_scatter(1)
            issue_idx(b, 3, 0)

            def triple(t, _):
                for j in range(3):
                    k = 2 + 3 * t + j
                    p = (2 + j) % 3
                    pn = (p + 1) % 3
                    pv = (p + 2) % 3
                    wait_gather(p)
                    wait_idx(b, k + 1, pn)
                    adjust_src(b, pn)
                    issue_gather(pn)
                    scale(p, k)
                    wait_scatter(pv)
                    issue_scatter(p)
                    issue_idx(b, k + 2, pv)
                return 0
            lax.fori_loop(0, (n_chunks - 2) // 3, triple, 0)

            wait_gather(n_chunks % 3)
            wait_idx(b, n_chunks + 1, (n_chunks + 1) % 3)
            wait_scatter((n_chunks - 1) % 3)
            plsc.subcore_barrier()

            def wb(nrows):
                base = sid * RPS_MAIN
                pltpu.sync_copy(
                    accum_sh.at[pl.ds(base, nrows)],
                    out_hbm.at[pl.ds(b * n_rows_out + base, nrows)])
            pl.when(sid < NUM_SUBCORES - 1)(lambda: wb(RPS_MAIN))
            pl.when(sid == NUM_SUBCORES - 1)(lambda: wb(RPS_LAST))
            plsc.subcore_barrier()

    return sc_kernel


def _group_mat(c):
    per = c // G
    i = lax.broadcasted_iota(jnp.int32, (c, c), 0) // per
    j = lax.broadcasted_iota(jnp.int32, (c, c), 1) // per
    return (i == j).astype(jnp.float32)


def _gn_scale_bias(x2d, gamma, beta, c):
    n = x2d.shape[0] * (c // G)
    s = jnp.sum(x2d, axis=0, keepdims=True)
    ss = jnp.sum(x2d * x2d, axis=0, keepdims=True)
    m = _group_mat(c)
    gs = jnp.dot(s, m, preferred_element_type=jnp.float32)
    gss = jnp.dot(ss, m, preferred_element_type=jnp.float32)
    mean = gs / n
    var = gss / n - mean * mean
    inv = lax.rsqrt(var + EPS)
    a = inv * gamma
    bb = beta - mean * a
    return a, bb


def _tc1_body(xu_ref, w1_ref, g1_ref, b1_ref, g2_ref, b2_ref, out_ref):
    halves = []
    for i in range(2):
        xb = xu_ref[i]
        a1, c1 = _gn_scale_bias(xb, g1_ref[...], b1_ref[...], CIN)
        t = jnp.maximum(xb * a1 + c1, 0.0)
        h = jnp.dot(t, w1_ref[0], preferred_element_type=jnp.float32)
        a2, c2 = _gn_scale_bias(h, g2_ref[...], b2_ref[...], CMID)
        halves.append(jnp.maximum(h * a2 + c2, 0.0))
    out_ref[0] = jnp.concatenate(halves, axis=1)


def _tc3_body(h2p_ref, aggp_ref, xu_ref, w20_ref, w21_ref, w30_ref,
              g3_ref, b3_ref, out_ref):
    h2p = h2p_ref[0]
    aggp = aggp_ref[0]
    for i in range(2):
        h2 = h2p[:, i * CMID:(i + 1) * CMID]
        agg = aggp[:, i * CMID:(i + 1) * CMID]
        o2 = (jnp.dot(h2, w20_ref[0], preferred_element_type=jnp.float32)
              + jnp.dot(agg, w21_ref[0], preferred_element_type=jnp.float32))
        a3, c3 = _gn_scale_bias(o2, g3_ref[...], b3_ref[...], CMID)
        h3 = jnp.maximum(o2 * a3 + c3, 0.0)
        out_ref[i] = (jnp.dot(h3, w30_ref[0],
                              preferred_element_type=jnp.float32)
                      + xu_ref[i])


def _batch_spec(n, c):
    return pl.BlockSpec((1, n, c), lambda b: (b, 0, 0))


def _full_spec(shape):
    nd = len(shape)
    return pl.BlockSpec(shape, lambda b: (0,) * nd)


def _ring_chunks(n):
    k = _cdiv(n, NUM_SUBCORES * CH)
    while k % 3 != 2:
        k += 1
    return k

_make_sc_scatter = functools.lru_cache(maxsize=None)(_make_sc_scatter)


@jax.jit
def kernel(x, up_row, up_col, up_val, A_edge_index, A_norm,
           W1, W2, W3, g1, b1, g2, b2, g3, b3):
    xu_flat = _make_sc_scatter(NC, NF, CIN, NNZ, B)(
        x.reshape(B * NC, CIN), up_col.astype(jnp.int32),
        up_row.astype(jnp.int32), up_val)
    xu = xu_flat.reshape(B, NF, CIN)

    h2p = pl.pallas_call(
        _tc1_body,
        grid=(2,),
        in_specs=[
            pl.BlockSpec((2, NF, CIN), lambda p: (p, 0, 0)),
            _full_spec((1, CIN, CMID)),
            _full_spec((1, CIN)), _full_spec((1, CIN)),
            _full_spec((1, CMID)), _full_spec((1, CMID)),
        ],
        out_specs=pl.BlockSpec((1, NF, 2 * CMID), lambda p: (p, 0, 0)),
        out_shape=jax.ShapeDtypeStruct((2, NF, 2 * CMID), jnp.float32),
        compiler_params=pltpu.CompilerParams(
            vmem_limit_bytes=100 * 1024 * 1024),
    )(xu, W1, g1.reshape(1, CIN), b1.reshape(1, CIN),
      g2.reshape(1, CMID), b2.reshape(1, CMID))

    aggp_flat = _make_sc_scatter(NF, NF, 2 * CMID, E, 2)(
        h2p.reshape(2 * NF, 2 * CMID), A_edge_index[0].astype(jnp.int32),
        A_edge_index[1].astype(jnp.int32), A_norm)
    aggp = aggp_flat.reshape(2, NF, 2 * CMID)

    out = pl.pallas_call(
        _tc3_body,
        grid=(2,),
        in_specs=[
            pl.BlockSpec((1, NF, 2 * CMID), lambda p: (p, 0, 0)),
            pl.BlockSpec((1, NF, 2 * CMID), lambda p: (p, 0, 0)),
            pl.BlockSpec((2, NF, CIN), lambda p: (p, 0, 0)),
            _full_spec((1, CMID, CMID)),
            _full_spec((1, CMID, CMID)),
            _full_spec((1, CMID, COUT)),
            _full_spec((1, CMID)), _full_spec((1, CMID)),
        ],
        out_specs=pl.BlockSpec((2, NF, COUT), lambda p: (p, 0, 0)),
        out_shape=jax.ShapeDtypeStruct((B, NF, COUT), jnp.float32),
        compiler_params=pltpu.CompilerParams(
            vmem_limit_bytes=100 * 1024 * 1024),
    )(h2p, aggp, xu, W2[0:1], W2[1:2], W3,
      g3.reshape(1, CMID), b3.reshape(1, CMID))
    return out


# speedup vs baseline: 1.0097x; 1.0097x over previous
import functools

import jax
import jax.numpy as jnp
from jax import lax
from jax.experimental import pallas as pl
from jax.experimental.pallas import tpu as pltpu
from jax.experimental.pallas import tpu_sc as plsc

B = 4
NC = 2500
NF = 10000
CIN = 128
COUT = 128
CMID = 64
E = 320000
NNZ = 30000
G = 32
EPS = 1e-5

NUM_CORES = 2
NUM_SUBCORES = 16
CH = 128
RPS_MAIN = 632
RPS_LAST = NF - (NUM_SUBCORES - 1) * RPS_MAIN


def _cdiv(a, b):
    return (a + b - 1) // b


_GDN = lax.GatherDimensionNumbers(
    offset_dims=(), collapsed_slice_dims=(0,), start_index_map=(0,))


def _splat_lane(vec, e):
    idx = (lax.iota(jnp.int32, 16) * 0 + e).reshape(16, 1)
    return lax.gather(vec, idx, dimension_numbers=_GDN,
                      slice_sizes=(1,),
                      mode=lax.GatherScatterMode.PROMISE_IN_BOUNDS)


def _make_sc_scatter(n_rows_tab, n_rows_out, c, n_edges, n_packs):
    assert n_edges % 8 == 0 and n_edges >= CH
    n_chunks = _ring_chunks(n_edges)
    assert n_chunks % 3 == 2
    e_per_s = n_chunks * CH
    rounds = n_packs // NUM_CORES
    mesh = plsc.VectorSubcoreMesh(core_axis_name="c", subcore_axis_name="s")

    @functools.partial(
        pl.kernel,
        out_type=jax.ShapeDtypeStruct((n_packs * n_rows_out, c), jnp.float32),
        mesh=mesh,
        scratch_types=[
            pltpu.VMEM((3, CH), jnp.int32),
            pltpu.VMEM((3, CH), jnp.int32),
            pltpu.VMEM((3, CH), jnp.float32),
            pltpu.VMEM((3, CH, c), jnp.float32),
            pltpu.VMEM_SHARED((n_rows_out, c), jnp.float32),
            pltpu.SemaphoreType.DMA,
            pltpu.SemaphoreType.DMA,
            pltpu.SemaphoreType.DMA,
            pltpu.SemaphoreType.DMA,
            pltpu.SemaphoreType.DMA,
            pltpu.SemaphoreType.DMA,
            pltpu.SemaphoreType.DMA,
            pltpu.SemaphoreType.DMA,
            pltpu.SemaphoreType.DMA,
        ],
    )
    def sc_kernel(tab_hbm, src_hbm, dst_hbm, val_hbm, out_hbm,
                  sidx_v, didx_v, val_s, msg_v, accum_sh,
                  sg0, sg1, sg2, si0, si1, si2, ss0, ss1, ss2):
        cid = lax.axis_index("c")
        sid = lax.axis_index("s")
        sg = (sg0, sg1, sg2)
        si = (si0, si1, si2)
        ss = (ss0, ss1, ss2)

        zeros16 = jnp.zeros((16,), jnp.float32)

        def zero_msg0(i, _):
            for j in range(c // 16):
                msg_v[0, i, pl.ds(j * 16, 16)] = zeros16
            return 0

        def init_slice(nrows):
            base = sid * RPS_MAIN
            for t in range(_cdiv(nrows, CH)):
                rows = min(CH, nrows - t * CH)
                pltpu.sync_copy(
                    msg_v.at[0].at[pl.ds(0, rows)],
                    accum_sh.at[pl.ds(base + t * CH, rows)])

        def chunk_base(k):
            off = sid * e_per_s + k * CH
            return off, jnp.minimum(off, n_edges - CH)

        def issue_idx(b, k, p):
            del b
            _, bc = chunk_base(k)
            pltpu.async_copy(src_hbm.at[pl.ds(bc, CH)], sidx_v.at[p], si[p])
            pltpu.async_copy(dst_hbm.at[pl.ds(bc, CH)], didx_v.at[p], si[p])
            pltpu.async_copy(val_hbm.at[pl.ds(bc, CH)], val_s.at[p], si[p])

        def wait_idx(b, k, p):
            del b
            _, bc = chunk_base(k)
            pltpu.make_async_copy(src_hbm.at[pl.ds(bc, CH)],
                                  sidx_v.at[p], si[p]).wait()
            pltpu.make_async_copy(dst_hbm.at[pl.ds(bc, CH)],
                                  didx_v.at[p], si[p]).wait()
            pltpu.make_async_copy(val_hbm.at[pl.ds(bc, CH)],
                                  val_s.at[p], si[p]).wait()

        def adjust_src(b, p):
            offv = b * n_rows_tab
            for g in range(CH // 16):
                sidx_v[p, pl.ds(g * 16, 16)] = (
                    sidx_v[p, pl.ds(g * 16, 16)] + offv)

        def issue_gather(p):
            pltpu.async_copy(tab_hbm.at[sidx_v.at[p]], msg_v.at[p], sg[p])

        def wait_gather(p):
            pltpu.make_async_copy(tab_hbm.at[sidx_v.at[p]], msg_v.at[p],
                                  sg[p]).wait()

        def issue_scatter(p):
            pltpu.async_copy(msg_v.at[p], accum_sh.at[didx_v.at[p]], ss[p],
                             add=True)

        def wait_scatter(p):
            pltpu.make_async_copy(msg_v.at[p], accum_sh.at[didx_v.at[p]],
                                  ss[p]).wait()

        def scale(p, k):
            off, bc = chunk_base(k)
            head = off - bc

            def body(g2, _):
                valv = val_s[p, pl.ds(g2 * 16, 16)]
                lane = lax.iota(jnp.int32, 16) + (g2 * 16 - head)
                valv = jnp.where(lane >= 0, valv, 0.0)
                for e2 in range(16):
                    v = _splat_lane(valv, e2)
                    row = g2 * 16 + e2
                    for j in range(c // 16):
                        msg_v[p, row, pl.ds(j * 16, 16)] = (
                            msg_v[p, row, pl.ds(j * 16, 16)] * v)
                return 0
            lax.fori_loop(0, CH // 16, body, 0, unroll=4)

        for r in range(rounds):
            b = cid + NUM_CORES * r

            lax.fori_loop(0, CH, zero_msg0, 0)
            pl.when(sid < NUM_SUBCORES - 1)(
                lambda: init_slice(RPS_MAIN))
            pl.when(sid == NUM_SUBCORES - 1)(
                lambda: init_slice(RPS_LAST))
            plsc.subcore_barrier()

            issue_idx(b, 0, 0)
            issue_idx(b, 1, 1)
            wait_idx(b, 0, 0)
            adjust_src(b, 0)
            issue_gather(0)

            wait_gather(0)
            wait_idx(b, 1, 1)
            adjust_src(b, 1)
            issue_gather(1)
            scale(0, 0)
            issue_scatter(0)
            issue_idx(b, 2, 2)

            wait_gather(1)
            wait_idx(b, 2, 2)
            adjust_src(b, 2)
            issue_gather(2)
            scale(1, 1)
            issue_scatter(1)
            wait_scatter(0)
            issue_idx(b, 3, 0)

            def triple(t, _):
                for j in range(3):
                    k = 2 + 3 * t + j
                    p = (2 + j) % 3
                    pn = (p + 1) % 3
                    pv = (p + 2) % 3
                    wait_gather(p)
                    wait_idx(b, k + 1, pn)
                    adjust_src(b, pn)
                    issue_gather(pn)
                    scale(p, k)
                    issue_scatter(p)
                    wait_scatter(pv)
                    issue_idx(b, k + 2, pv)
                return 0
            lax.fori_loop(0, (n_chunks - 2) // 3, triple, 0)

            wait_gather(n_chunks % 3)
            wait_idx(b, n_chunks + 1, (n_chunks + 1) % 3)
            wait_scatter((n_chunks - 1) % 3)
            plsc.subcore_barrier()

            def wb(nrows):
                base = sid * RPS_MAIN
                pltpu.sync_copy(
                    accum_sh.at[pl.ds(base, nrows)],
                    out_hbm.at[pl.ds(b * n_rows_out + base, nrows)])
            pl.when(sid < NUM_SUBCORES - 1)(lambda: wb(RPS_MAIN))
            pl.when(sid == NUM_SUBCORES - 1)(lambda: wb(RPS_LAST))
            plsc.subcore_barrier()

    return sc_kernel


def _group_mat(c):
    per = c // G
    i = lax.broadcasted_iota(jnp.int32, (c, c), 0) // per
    j = lax.broadcasted_iota(jnp.int32, (c, c), 1) // per
    return (i == j).astype(jnp.float32)


def _gn_scale_bias(x2d, gamma, beta, c):
    n = x2d.shape[0] * (c // G)
    s = jnp.sum(x2d, axis=0, keepdims=True)
    ss = jnp.sum(x2d * x2d, axis=0, keepdims=True)
    m = _group_mat(c)
    gs = jnp.dot(s, m, preferred_element_type=jnp.float32)
    gss = jnp.dot(ss, m, preferred_element_type=jnp.float32)
    mean = gs / n
    var = gss / n - mean * mean
    inv = lax.rsqrt(var + EPS)
    a = inv * gamma
    bb = beta - mean * a
    return a, bb


def _tc1_body(xu_ref, w1_ref, g1_ref, b1_ref, g2_ref, b2_ref, out_ref):
    halves = []
    for i in range(2):
        xb = xu_ref[i]
        a1, c1 = _gn_scale_bias(xb, g1_ref[...], b1_ref[...], CIN)
        t = jnp.maximum(xb * a1 + c1, 0.0)
        h = jnp.dot(t, w1_ref[0], preferred_element_type=jnp.float32)
        a2, c2 = _gn_scale_bias(h, g2_ref[...], b2_ref[...], CMID)
        halves.append(jnp.maximum(h * a2 + c2, 0.0))
    out_ref[0] = jnp.concatenate(halves, axis=1)


def _tc3_body(h2p_ref, aggp_ref, xu_ref, w20_ref, w21_ref, w30_ref,
              g3_ref, b3_ref, out_ref):
    h2p = h2p_ref[0]
    aggp = aggp_ref[0]
    for i in range(2):
        h2 = h2p[:, i * CMID:(i + 1) * CMID]
        agg = aggp[:, i * CMID:(i + 1) * CMID]
        o2 = (jnp.dot(h2, w20_ref[0], preferred_element_type=jnp.float32)
              + jnp.dot(agg, w21_ref[0], preferred_element_type=jnp.float32))
        a3, c3 = _gn_scale_bias(o2, g3_ref[...], b3_ref[...], CMID)
        h3 = jnp.maximum(o2 * a3 + c3, 0.0)
        out_ref[i] = (jnp.dot(h3, w30_ref[0],
                              preferred_element_type=jnp.float32)
                      + xu_ref[i])


def _batch_spec(n, c):
    return pl.BlockSpec((1, n, c), lambda b: (b, 0, 0))


def _full_spec(shape):
    nd = len(shape)
    return pl.BlockSpec(shape, lambda b: (0,) * nd)


def _ring_chunks(n):
    k = _cdiv(n, NUM_SUBCORES * CH)
    while k % 3 != 2:
        k += 1
    return k

_make_sc_scatter = functools.lru_cache(maxsize=None)(_make_sc_scatter)


@jax.jit
def kernel(x, up_row, up_col, up_val, A_edge_index, A_norm,
           W1, W2, W3, g1, b1, g2, b2, g3, b3):
    xu_flat = _make_sc_scatter(NC, NF, CIN, NNZ, B)(
        x.reshape(B * NC, CIN), up_col.astype(jnp.int32),
        up_row.astype(jnp.int32), up_val)
    xu = xu_flat.reshape(B, NF, CIN)

    h2p = pl.pallas_call(
        _tc1_body,
        grid=(2,),
        in_specs=[
            pl.BlockSpec((2, NF, CIN), lambda p: (p, 0, 0)),
            _full_spec((1, CIN, CMID)),
            _full_spec((1, CIN)), _full_spec((1, CIN)),
            _full_spec((1, CMID)), _full_spec((1, CMID)),
        ],
        out_specs=pl.BlockSpec((1, NF, 2 * CMID), lambda p: (p, 0, 0)),
        out_shape=jax.ShapeDtypeStruct((2, NF, 2 * CMID), jnp.float32),
        compiler_params=pltpu.CompilerParams(
            vmem_limit_bytes=100 * 1024 * 1024),
    )(xu, W1, g1.reshape(1, CIN), b1.reshape(1, CIN),
      g2.reshape(1, CMID), b2.reshape(1, CMID))

    aggp_flat = _make_sc_scatter(NF, NF, 2 * CMID, E, 2)(
        h2p.reshape(2 * NF, 2 * CMID), A_edge_index[0].astype(jnp.int32),
        A_edge_index[1].astype(jnp.int32), A_norm)
    aggp = aggp_flat.reshape(2, NF, 2 * CMID)

    out = pl.pallas_call(
        _tc3_body,
        grid=(2,),
        in_specs=[
            pl.BlockSpec((1, NF, 2 * CMID), lambda p: (p, 0, 0)),
            pl.BlockSpec((1, NF, 2 * CMID), lambda p: (p, 0, 0)),
            pl.BlockSpec((2, NF, CIN), lambda p: (p, 0, 0)),
            _full_spec((1, CMID, CMID)),
            _full_spec((1, CMID, CMID)),
            _full_spec((1, CMID, COUT)),
            _full_spec((1, CMID)), _full_spec((1, CMID)),
        ],
        out_specs=pl.BlockSpec((2, NF, COUT), lambda p: (p, 0, 0)),
        out_shape=jax.ShapeDtypeStruct((B, NF, COUT), jnp.float32),
        compiler_params=pltpu.CompilerParams(
            vmem_limit_bytes=100 * 1024 * 1024),
    )(h2p, aggp, xu, W2[0:1], W2[1:2], W3,
      g3.reshape(1, CMID), b3.reshape(1, CMID))
    return out
